# baseline (speedup 1.0000x reference)
import functools
import jax
import jax.numpy as jnp
from jax import lax
from jax.experimental import pallas as pl
from jax.experimental.pallas import tpu as pltpu, tpu_sc as plsc

_NUM_CLASSES = 1000
_FEAT = 100
_OUT_COLS = _NUM_CLASSES + _FEAT
_CHUNK = 32


def kernel(x):
    batch, feat = x.shape
    info = plsc.get_sparse_core_info()
    nc, ns = info.num_cores, info.num_subcores
    nw = nc * ns
    rows_per_w = batch // nw
    nchunks = rows_per_w // _CHUNK
    mesh = plsc.VectorSubcoreMesh(core_axis_name="c", subcore_axis_name="s",
                                  num_cores=nc)

    @functools.partial(
        pl.kernel,
        mesh=mesh,
        compiler_params=pltpu.CompilerParams(use_tc_tiling_on_sc=False,
                                             needs_layout_passes=False),
        out_type=jax.ShapeDtypeStruct((batch, _OUT_COLS), jnp.float32),
        scratch_types=[
            pltpu.VMEM((_CHUNK, _OUT_COLS), jnp.float32),
            pltpu.VMEM((_CHUNK, _OUT_COLS), jnp.float32),
            pltpu.VMEM((_CHUNK,), jnp.int32),
            pltpu.VMEM((_CHUNK,), jnp.int32),
            pltpu.SemaphoreType.DMA,
            pltpu.SemaphoreType.DMA,
        ],
    )
    def run(x_hbm, out_hbm, buf0, buf1, sel0, sel1, sem0, sem1):
        wid = lax.axis_index("s") * nc + lax.axis_index("c")
        base = wid * rows_per_w
        iota16 = lax.iota(jnp.int32, 16)
        zeros_f = jnp.zeros((16,), jnp.float32)
        ones_f = jnp.ones((16,), jnp.float32)
        zeros_i = jnp.zeros((16,), jnp.int32)
        col0 = jnp.full((16,), _NUM_CLASSES, jnp.int32)

        def zrow(r, carry):
            for c in range(63):
                buf0[r, pl.ds(c * 16, 16)] = zeros_f
                buf1[r, pl.ds(c * 16, 16)] = zeros_f
            return carry
        lax.fori_loop(0, _CHUNK, zrow, 0)
        for j in range(_CHUNK // 16):
            sel0[pl.ds(j * 16, 16)] = zeros_i
            sel1[pl.ds(j * 16, 16)] = zeros_i

        def chunk(ci, buf, sel_ref, sem, do_wait):
            rowbase = base + ci * _CHUNK
            dst = out_hbm.at[pl.ds(rowbase, _CHUNK), :]
            if do_wait:
                pltpu.make_async_copy(buf, dst, sem).wait()
            for j in range(_CHUNK // 16):
                rows = iota16 + (j * 16)
                prev = sel_ref[pl.ds(j * 16, 16)]
                plsc.store_scatter(buf, [rows, prev], zeros_f)
            pltpu.sync_copy(x_hbm.at[pl.ds(rowbase, _CHUNK), :],
                            buf.at[:, pl.ds(_NUM_CLASSES, feat)])
            for j in range(_CHUNK // 16):
                rows = iota16 + (j * 16)
                vals = plsc.load_gather(buf, [rows, col0])
                sel = vals.astype(jnp.int32)
                sel_ref[pl.ds(j * 16, 16)] = sel
                plsc.store_scatter(buf, [rows, sel], ones_f)
            pltpu.make_async_copy(buf, dst, sem).start()

        chunk(0, buf0, sel0, sem0, False)
        chunk(1, buf1, sel1, sem1, False)

        def loop_body(g, carry):
            chunk(2 * g, buf0, sel0, sem0, True)
            chunk(2 * g + 1, buf1, sel1, sem1, True)
            return carry
        lax.fori_loop(1, nchunks // 2, loop_body, 0)

        last0 = base + (nchunks - 2) * _CHUNK
        last1 = base + (nchunks - 1) * _CHUNK
        pltpu.make_async_copy(buf0, out_hbm.at[pl.ds(last0, _CHUNK), :], sem0).wait()
        pltpu.make_async_copy(buf1, out_hbm.at[pl.ds(last1, _CHUNK), :], sem1).wait()

    return run(x)
